# baseline (speedup 1.0000x reference)
import functools

import jax
import jax.numpy as jnp
from jax import lax
from jax.experimental import pallas as pl
from jax.experimental.pallas import tpu as pltpu
from jax.experimental.pallas import tpu_sc as plsc

H = 1024
W = 1024
C = 32
N = 524288

_INFO = plsc.get_sparse_core_info()
NC = _INFO.num_cores
NS = _INFO.num_subcores
NW = NC * NS
PTS = N // NW
P = 128
NCHUNK = PTS // P
NG = P // 16
NBUF = 4
NB = N // 128
CB = C // 8

_mesh = plsc.VectorSubcoreMesh(core_axis_name="c", subcore_axis_name="s")

UNITS = H * 8
UPW = UNITS // NW


@functools.partial(
    pl.kernel,
    mesh=_mesh,
    out_type=jax.ShapeDtypeStruct((H * W, C), jnp.bfloat16),
    scratch_types=[
        pltpu.VMEM((4, C, 129), jnp.float32),
        pltpu.VMEM((4, 128, C), jnp.bfloat16),
        [pltpu.SemaphoreType.DMA] * 4,
        pltpu.SemaphoreType.DMA,
    ],
    compiler_params=pltpu.CompilerParams(
        use_tc_tiling_on_sc=False, needs_layout_passes=False),
)
def _detile(a_hbm, out_hbm, t_v, o_v, rsems, wsem):
    wid = lax.axis_index("s") * NC + lax.axis_index("c")
    u0 = wid * UPW

    lanes = lax.iota(jnp.int32, 16)

    def fire_reads(u, b):
        h = u // 8
        wb = u % 8
        for cb in range(4):
            pltpu.async_copy(
                a_hbm.at[h, cb, wb],
                t_v.at[b, pl.ds(cb * 8, 8), pl.ds(0, 128)], rsems[b])

    def wait_reads(b):
        for cb in range(4):
            pltpu.make_async_copy(
                a_hbm.at[0, cb, 0],
                t_v.at[b, pl.ds(cb * 8, 8), pl.ds(0, 128)], rsems[b]).wait()

    def transpose(b):
        def row4(r, carry):
            for j in range(4):
                wm = r * 4 + j
                wmv = jnp.full((16,), wm, jnp.int32)
                ge = plsc.load_gather(t_v.at[b], [2 * lanes, wmv])
                go = plsc.load_gather(t_v.at[b], [2 * lanes + 1, wmv])
                o_v[b, wm, :] = plsc.pack(
                    ge, go, format=plsc.PackFormat.INTERLEAVED)
            return carry
        lax.fori_loop(0, 32, row4, 0)

    def fire_write(u, b):
        h = u // 8
        wb = u % 8
        pltpu.async_copy(
            o_v.at[b], out_hbm.at[pl.ds(h * 1024 + wb * 128, 128)], wsem)

    def drain_write():
        pltpu.make_async_copy(
            o_v.at[0], out_hbm.at[pl.ds(0, 128)], wsem).wait()

    for b in range(3):
        fire_reads(u0 + b, b)

    @pl.loop(0, UPW, step=4)
    def _units(k):
        for b in range(4):
            u = u0 + k + b

            @pl.when(k + b + 3 < UPW)
            def _():
                fire_reads(u + 3, (b + 3) % 4)

            wait_reads(b)

            @pl.when(k + b >= 4)
            def _():
                drain_write()

            transpose(b)
            fire_write(u, b)

    for _ in range(4):
        drain_write()


@functools.partial(
    pl.kernel,
    mesh=_mesh,
    out_type=jax.ShapeDtypeStruct((CB, NB, 8, 128), jnp.float32),
    scratch_types=[
        pltpu.VMEM((PTS,), jnp.float32),
        pltpu.VMEM((PTS,), jnp.float32),
        pltpu.VMEM((NBUF, P), jnp.float32),
        pltpu.VMEM((NBUF, P), jnp.float32),
        pltpu.VMEM((NBUF * 4, P), jnp.int32),
        pltpu.VMEM((NBUF * 4, P, C), jnp.bfloat16),
        pltpu.VMEM((NBUF, C, P + 1), jnp.float32),
        [pltpu.SemaphoreType.DMA] * NBUF,
        pltpu.SemaphoreType.DMA,
    ],
    compiler_params=pltpu.CompilerParams(
        use_tc_tiling_on_sc=False, needs_layout_passes=False),
)
def _grid_sample(cx_hbm, cy_hbm, tab_hbm, out_hbm,
                 cx_v, cy_v, wx_v, wy_v, i_v, r_v, ob_v, gsems, ssem):
    wid = lax.axis_index("s") * NC + lax.axis_index("c")
    tile_base = wid * PTS

    lanes = lax.iota(jnp.int32, 16)
    ch_idx = [2 * lanes, 2 * lanes + 1]

    def idx_and_fire(c, b):
        for g in range(NG):
            s = c * P + g * 16
            sl = pl.ds(s, 16)
            lx = (cx_v[sl] + 0.5) * (W - 1.0)
            ly = (cy_v[sl] + 0.5) * (H - 1.0)
            x0 = lx.astype(jnp.int32)
            y0 = ly.astype(jnp.int32)
            gs = pl.ds(g * 16, 16)
            wx_v[b, gs] = lx - x0.astype(jnp.float32)
            wy_v[b, gs] = ly - y0.astype(jnp.float32)
            x1 = jnp.minimum(x0 + 1, W - 1)
            y0w = y0 * W
            y1w = jnp.minimum(y0 + 1, H - 1) * W
            i_v[4 * b + 0, gs] = y0w + x0
            i_v[4 * b + 1, gs] = y0w + x1
            i_v[4 * b + 2, gs] = y1w + x0
            i_v[4 * b + 3, gs] = y1w + x1
        for k in range(4):
            pltpu.async_copy(
                tab_hbm.at[i_v.at[4 * b + k]], r_v.at[4 * b + k], gsems[b])

    def wait_gathers(b):
        for k in range(4):
            pltpu.make_async_copy(
                tab_hbm.at[i_v.at[4 * b + k]], r_v.at[4 * b + k],
                gsems[b]).wait()

    def interp(b):
        def group(g, carry):
            s = g * 16
            wxv = wx_v[b, pl.ds(s, 16)]
            wyv = wy_v[b, pl.ds(s, 16)]
            for j in range(16):
                p = s + j
                wxj = jnp.full((16,), wxv[j])
                wyj = jnp.full((16,), wyv[j])
                pv = jnp.full((16,), p, jnp.int32)
                g00 = plsc.unpack(r_v[4 * b + 0, p, :],
                                  format=plsc.PackFormat.INTERLEAVED)
                g01 = plsc.unpack(r_v[4 * b + 1, p, :],
                                  format=plsc.PackFormat.INTERLEAVED)
                g10 = plsc.unpack(r_v[4 * b + 2, p, :],
                                  format=plsc.PackFormat.INTERLEAVED)
                g11 = plsc.unpack(r_v[4 * b + 3, p, :],
                                  format=plsc.PackFormat.INTERLEAVED)
                for h in range(2):
                    top = g00[h] + wxj * (g01[h] - g00[h])
                    bot = g10[h] + wxj * (g11[h] - g10[h])
                    plsc.store_scatter(
                        ob_v.at[b], [ch_idx[h], pv],
                        top + wyj * (bot - top))
            return carry
        lax.fori_loop(0, NG, group, 0)

    def fire_store(c, b):
        nb = wid * NCHUNK + c
        for cb in range(CB):
            pltpu.async_copy(
                ob_v.at[b, pl.ds(cb * 8, 8), pl.ds(0, P)],
                out_hbm.at[cb, nb], ssem)

    def drain_store():
        for cb in range(CB):
            pltpu.make_async_copy(
                ob_v.at[0, pl.ds(cb * 8, 8), pl.ds(0, P)],
                out_hbm.at[cb, wid * NCHUNK], ssem).wait()

    pltpu.sync_copy(cx_hbm.at[pl.ds(tile_base, PTS)], cx_v)
    pltpu.sync_copy(cy_hbm.at[pl.ds(tile_base, PTS)], cy_v)

    for b in range(NBUF - 1):
        idx_and_fire(b, b)

    @pl.loop(0, NCHUNK, step=NBUF)
    def _outer(i):
        for b in range(NBUF):
            c = i + b

            @pl.when(c + NBUF - 1 < NCHUNK)
            def _():
                idx_and_fire(c + NBUF - 1, (b + NBUF - 1) % NBUF)

            @pl.when(c >= NBUF)
            def _():
                drain_store()

            wait_gathers(b)
            interp(b)
            fire_store(c, b)

    for _ in range(NBUF):
        drain_store()


def kernel(x, features):
    xt = x.reshape(N, 2).T
    a = features.reshape(H, 8, 128, 4, 8).transpose(0, 3, 1, 4, 2)
    tab = _detile(a)
    raw = _grid_sample(xt[0], xt[1], tab)
    out = (raw.transpose(0, 2, 1, 3)
              .reshape(C, N)
              .T.reshape(1, N, C))
    return out


# speedup vs baseline: 1.0028x; 1.0028x over previous
import functools

import jax
import jax.numpy as jnp
from jax import lax
from jax.experimental import pallas as pl
from jax.experimental.pallas import tpu as pltpu
from jax.experimental.pallas import tpu_sc as plsc

H = 1024
W = 1024
C = 32
N = 524288

_INFO = plsc.get_sparse_core_info()
NC = _INFO.num_cores
NS = _INFO.num_subcores
NW = NC * NS
PTS = N // NW
P = 128
NCHUNK = PTS // P
NG = P // 16
NBUF = 4
NB = N // 128
CB = C // 8

_mesh = plsc.VectorSubcoreMesh(core_axis_name="c", subcore_axis_name="s")

UNITS = H * 8
UPW = UNITS // NW


@functools.partial(
    pl.kernel,
    mesh=_mesh,
    out_type=jax.ShapeDtypeStruct((H * W, C), jnp.bfloat16),
    scratch_types=[
        pltpu.VMEM((4, C, 129), jnp.float32),
        pltpu.VMEM((4, 128, C), jnp.bfloat16),
        [pltpu.SemaphoreType.DMA] * 4,
        pltpu.SemaphoreType.DMA,
    ],
    compiler_params=pltpu.CompilerParams(
        use_tc_tiling_on_sc=False, needs_layout_passes=False),
)
def _detile(a_hbm, out_hbm, t_v, o_v, rsems, wsem):
    wid = lax.axis_index("s") * NC + lax.axis_index("c")
    u0 = wid * UPW

    lanes = lax.iota(jnp.int32, 16)

    def fire_reads(u, b):
        h = u // 8
        wb = u % 8
        for cb in range(4):
            pltpu.async_copy(
                a_hbm.at[h, cb, wb],
                t_v.at[b, pl.ds(cb * 8, 8), pl.ds(0, 128)], rsems[b])

    def wait_reads(b):
        for cb in range(4):
            pltpu.make_async_copy(
                a_hbm.at[0, cb, 0],
                t_v.at[b, pl.ds(cb * 8, 8), pl.ds(0, 128)], rsems[b]).wait()

    def transpose(b):
        def row(wm, carry):
            wmv = jnp.full((16,), wm, jnp.int32)
            ge = plsc.load_gather(t_v.at[b], [2 * lanes, wmv])
            go = plsc.load_gather(t_v.at[b], [2 * lanes + 1, wmv])
            o_v[b, wm, :] = plsc.pack(
                ge, go, format=plsc.PackFormat.INTERLEAVED)
            return carry
        lax.fori_loop(0, 128, row, 0)

    def fire_write(u, b):
        h = u // 8
        wb = u % 8
        pltpu.async_copy(
            o_v.at[b], out_hbm.at[pl.ds(h * 1024 + wb * 128, 128)], wsem)

    def drain_write():
        pltpu.make_async_copy(
            o_v.at[0], out_hbm.at[pl.ds(0, 128)], wsem).wait()

    for b in range(3):
        fire_reads(u0 + b, b)

    @pl.loop(0, UPW, step=4)
    def _units(k):
        for b in range(4):
            u = u0 + k + b

            @pl.when(k + b + 3 < UPW)
            def _():
                fire_reads(u + 3, (b + 3) % 4)

            wait_reads(b)

            @pl.when(k + b >= 4)
            def _():
                drain_write()

            transpose(b)
            fire_write(u, b)

    for _ in range(4):
        drain_write()


@functools.partial(
    pl.kernel,
    mesh=_mesh,
    out_type=jax.ShapeDtypeStruct((CB, NB, 8, 128), jnp.float32),
    scratch_types=[
        pltpu.VMEM((PTS,), jnp.float32),
        pltpu.VMEM((PTS,), jnp.float32),
        pltpu.VMEM((NBUF, P), jnp.float32),
        pltpu.VMEM((NBUF, P), jnp.float32),
        pltpu.VMEM((NBUF * 4, P), jnp.int32),
        pltpu.VMEM((NBUF * 4, P, C), jnp.bfloat16),
        pltpu.VMEM((NBUF, C, P + 1), jnp.float32),
        [pltpu.SemaphoreType.DMA] * NBUF,
        pltpu.SemaphoreType.DMA,
    ],
    compiler_params=pltpu.CompilerParams(
        use_tc_tiling_on_sc=False, needs_layout_passes=False),
)
def _grid_sample(cx_hbm, cy_hbm, tab_hbm, out_hbm,
                 cx_v, cy_v, wx_v, wy_v, i_v, r_v, ob_v, gsems, ssem):
    wid = lax.axis_index("s") * NC + lax.axis_index("c")
    tile_base = wid * PTS

    lanes = lax.iota(jnp.int32, 16)
    ch_idx = [2 * lanes, 2 * lanes + 1]

    def idx_and_fire(c, b):
        for g in range(NG):
            s = c * P + g * 16
            sl = pl.ds(s, 16)
            lx = (cx_v[sl] + 0.5) * (W - 1.0)
            ly = (cy_v[sl] + 0.5) * (H - 1.0)
            x0 = lx.astype(jnp.int32)
            y0 = ly.astype(jnp.int32)
            gs = pl.ds(g * 16, 16)
            wx_v[b, gs] = lx - x0.astype(jnp.float32)
            wy_v[b, gs] = ly - y0.astype(jnp.float32)
            x1 = jnp.minimum(x0 + 1, W - 1)
            y0w = y0 * W
            y1w = jnp.minimum(y0 + 1, H - 1) * W
            i_v[4 * b + 0, gs] = y0w + x0
            i_v[4 * b + 1, gs] = y0w + x1
            i_v[4 * b + 2, gs] = y1w + x0
            i_v[4 * b + 3, gs] = y1w + x1
        for k in range(4):
            pltpu.async_copy(
                tab_hbm.at[i_v.at[4 * b + k]], r_v.at[4 * b + k], gsems[b])

    def wait_gathers(b):
        for k in range(4):
            pltpu.make_async_copy(
                tab_hbm.at[i_v.at[4 * b + k]], r_v.at[4 * b + k],
                gsems[b]).wait()

    def interp(b):
        def group(g, carry):
            s = g * 16
            wxv = wx_v[b, pl.ds(s, 16)]
            wyv = wy_v[b, pl.ds(s, 16)]
            for j in range(16):
                p = s + j
                wxj = jnp.full((16,), wxv[j])
                wyj = jnp.full((16,), wyv[j])
                pv = jnp.full((16,), p, jnp.int32)
                g00 = plsc.unpack(r_v[4 * b + 0, p, :],
                                  format=plsc.PackFormat.INTERLEAVED)
                g01 = plsc.unpack(r_v[4 * b + 1, p, :],
                                  format=plsc.PackFormat.INTERLEAVED)
                g10 = plsc.unpack(r_v[4 * b + 2, p, :],
                                  format=plsc.PackFormat.INTERLEAVED)
                g11 = plsc.unpack(r_v[4 * b + 3, p, :],
                                  format=plsc.PackFormat.INTERLEAVED)
                for h in range(2):
                    top = g00[h] + wxj * (g01[h] - g00[h])
                    bot = g10[h] + wxj * (g11[h] - g10[h])
                    plsc.store_scatter(
                        ob_v.at[b], [ch_idx[h], pv],
                        top + wyj * (bot - top))
            return carry
        lax.fori_loop(0, NG, group, 0)

    def fire_store(c, b):
        nb = wid * NCHUNK + c
        for cb in range(CB):
            pltpu.async_copy(
                ob_v.at[b, pl.ds(cb * 8, 8), pl.ds(0, P)],
                out_hbm.at[cb, nb], ssem)

    def drain_store():
        for cb in range(CB):
            pltpu.make_async_copy(
                ob_v.at[0, pl.ds(cb * 8, 8), pl.ds(0, P)],
                out_hbm.at[cb, wid * NCHUNK], ssem).wait()

    pltpu.sync_copy(cx_hbm.at[pl.ds(tile_base, PTS)], cx_v)
    pltpu.sync_copy(cy_hbm.at[pl.ds(tile_base, PTS)], cy_v)

    for b in range(NBUF - 1):
        idx_and_fire(b, b)

    @pl.loop(0, NCHUNK, step=NBUF)
    def _outer(i):
        for b in range(NBUF):
            c = i + b

            @pl.when(c + NBUF - 1 < NCHUNK)
            def _():
                idx_and_fire(c + NBUF - 1, (b + NBUF - 1) % NBUF)

            @pl.when(c >= NBUF)
            def _():
                drain_store()

            wait_gathers(b)
            interp(b)
            fire_store(c, b)

    for _ in range(NBUF):
        drain_store()


def kernel(x, features):
    xt = x.reshape(N, 2).T
    a = features.reshape(H, 8, 128, 4, 8).transpose(0, 3, 1, 4, 2)
    tab = _detile(a)
    raw = _grid_sample(xt[0], xt[1], tab)
    out = (raw.transpose(0, 2, 1, 3)
              .reshape(C, N)
              .T.reshape(1, N, C))
    return out
